# Initial kernel scaffold; baseline (speedup 1.0000x reference)
import jax
import jax.numpy as jnp
from jax.experimental import pallas as pl


def kernel(x, emb, edge_index, pedge_index, W_node, b_node, W_emb, b_emb, conv_init_w, conv_root_w, conv_bias, bn_gamma, bn_beta, W1, b1, W2, b2):
    raise NotImplementedError("write your pallas kernel here")


# speedup vs baseline: 5.1651x; 5.1651x over previous
import jax
import jax.numpy as jnp
from jax import lax
from jax.experimental import pallas as pl
from jax.experimental.pallas import tpu as pltpu
from jax.experimental.pallas import tpu_sc as plsc

_NC = 2
_NS = 16
_NW = _NC * _NS

_N = 50000
_E = 800000
_PE = 100000
_HID = 72
_HPAD = 128
_UPAD = 80

_NACC = 51200
_TPW = _NACC // _NS
_EPW = 25600
_EPAD = _EPW * _NW
_ERPW = _EPW // 128
_NCHUNK = 5

_GPAD = 204800
_GRPW = _GPAD // 128 // _NW

_BM = 2000
_NB = _N // _BM
_BPE = 2000
_NPB = _PE // _BPE


def _deg_body(col2, ones_hbm, zslab, out, cbuf, obuf, acc):
    c = lax.axis_index("c")
    s = lax.axis_index("s")
    wid = s * _NC + c
    pltpu.sync_copy(zslab, acc.at[pl.ds(s * _TPW, _TPW)])
    pltpu.sync_copy(ones_hbm, obuf)
    plsc.subcore_barrier()

    def blk(g, carry):
        base = wid * _ERPW + g * 8
        pltpu.sync_copy(col2.at[pl.ds(base, 8)], cbuf)
        for j in range(8):
            pltpu.sync_copy(obuf, acc.at[cbuf.at[j]], add=True)
        return carry

    lax.fori_loop(0, _ERPW // 8, blk, 0)
    plsc.subcore_barrier()
    pltpu.sync_copy(acc.at[pl.ds(s * _TPW, _TPW)],
                    out.at[c, pl.ds(s * _TPW, _TPW)])


def _sc_degree(col2, ones16, zslab):
    mesh = plsc.VectorSubcoreMesh(core_axis_name="c", subcore_axis_name="s")
    f = pl.kernel(
        _deg_body,
        out_type=jax.ShapeDtypeStruct((_NC, _NACC, 16), jnp.float32),
        mesh=mesh,
        compiler_params=pltpu.CompilerParams(use_tc_tiling_on_sc=False),
        scratch_types=[
            pltpu.VMEM((8, 128), jnp.int32),
            pltpu.VMEM((128, 16), jnp.float32),
            pltpu.VMEM_SHARED((_NACC, 16), jnp.float32),
        ],
    )
    return f(col2, ones16, zslab)


def _seg_body(p0, p1, p2, p3, p4, row2, col2, zslab, out,
              rbuf, cbuf, gbuf, acc, sem):
    c = lax.axis_index("c")
    s = lax.axis_index("s")
    wid = s * _NC + c
    ps = [p0, p1, p2, p3, p4]
    pltpu.sync_copy(zslab, acc.at[pl.ds(s * _TPW, _TPW)])
    plsc.subcore_barrier()
    for k in range(_NCHUNK):
        pk = ps[k]

        def blk(g, carry):
            base = wid * _ERPW + g * 8
            pltpu.sync_copy(row2.at[pl.ds(base, 8)], rbuf)
            pltpu.sync_copy(col2.at[pl.ds(base, 8)], cbuf)
            descs = [pltpu.async_copy(pk.at[rbuf.at[j]], gbuf.at[j], sem)
                     for j in range(8)]
            for d in descs:
                d.wait()
            for j in range(8):
                pltpu.sync_copy(gbuf.at[j], acc.at[cbuf.at[j]], add=True)
            return carry

        lax.fori_loop(0, _ERPW // 8, blk, 0)
        plsc.subcore_barrier()
        pltpu.sync_copy(acc.at[pl.ds(s * _TPW, _TPW)],
                        out.at[k, c, pl.ds(s * _TPW, _TPW)])
        if k + 1 < _NCHUNK:
            pltpu.sync_copy(zslab, acc.at[pl.ds(s * _TPW, _TPW)])
        plsc.subcore_barrier()


def _sc_segsum(ps, row2, col2, zslab):
    mesh = plsc.VectorSubcoreMesh(core_axis_name="c", subcore_axis_name="s")
    f = pl.kernel(
        _seg_body,
        out_type=jax.ShapeDtypeStruct((_NCHUNK, _NC, _NACC, 16), jnp.float32),
        mesh=mesh,
        compiler_params=pltpu.CompilerParams(use_tc_tiling_on_sc=False),
        scratch_types=[
            pltpu.VMEM((8, 128), jnp.int32),
            pltpu.VMEM((8, 128), jnp.int32),
            pltpu.VMEM((8, 128, 16), jnp.float32),
            pltpu.VMEM_SHARED((_NACC, 16), jnp.float32),
            pltpu.SemaphoreType.DMA,
        ],
    )
    return f(ps[0], ps[1], ps[2], ps[3], ps[4], row2, col2, zslab)


def _gat_body(uv, idx2, out, ibuf, gbuf, sem):
    c = lax.axis_index("c")
    s = lax.axis_index("s")
    wid = s * _NC + c

    def blk(g, carry):
        base = wid * _GRPW + g * 5
        pltpu.sync_copy(idx2.at[pl.ds(base, 5)], ibuf)
        descs = [pltpu.async_copy(uv.at[ibuf.at[j]], gbuf.at[j], sem)
                 for j in range(5)]
        for d in descs:
            d.wait()
        pltpu.sync_copy(gbuf, out.at[pl.ds(base, 5)])
        return carry

    lax.fori_loop(0, _GRPW // 5, blk, 0)


def _sc_gather(uv, idx2):
    mesh = plsc.VectorSubcoreMesh(core_axis_name="c", subcore_axis_name="s")
    f = pl.kernel(
        _gat_body,
        out_type=jax.ShapeDtypeStruct((_GPAD // 128, 128, _UPAD), jnp.float32),
        mesh=mesh,
        compiler_params=pltpu.CompilerParams(use_tc_tiling_on_sc=False),
        scratch_types=[
            pltpu.VMEM((5, 128), jnp.int32),
            pltpu.VMEM((5, 128, _UPAD), jnp.float32),
            pltpu.SemaphoreType.DMA,
        ],
    )
    return f(uv, idx2)


def _dinv_from_deg(dref):
    deg = dref[0, :, 0:1] + dref[1, :, 0:1]
    return jnp.where(deg > 0, lax.rsqrt(jnp.maximum(deg, 1e-12)), 0.0)


def _prep_body(x, emb, wn, bn, we, be, cw0, dref, h0, eo, *pouts):
    dinv = _dinv_from_deg(dref)
    hb = jnp.maximum(jnp.dot(x[...], wn[...],
                             preferred_element_type=jnp.float32) + bn[...], 0.0)
    eb = jnp.maximum(jnp.dot(emb[...], we[...],
                             preferred_element_type=jnp.float32) + be[...], 0.0)
    h0[...] = hb
    eo[...] = eb
    out0 = jnp.dot(hb, cw0[...], preferred_element_type=jnp.float32)
    for k in range(_NCHUNK):
        pouts[k][...] = dinv * out0[:, 16 * k:16 * (k + 1)]


def _tc_prep(x, emb, wn, bn, we, be, cw0, degslab):
    bn_ = pl.BlockSpec((_BM, _HPAD), lambda i: (i, 0))
    bw = pl.BlockSpec((_HPAD, _HPAD), lambda i: (0, 0))
    bb = pl.BlockSpec((1, _HPAD), lambda i: (0, 0))
    bd = pl.BlockSpec((2, _BM, 16), lambda i: (0, i, 0))
    bp = pl.BlockSpec((_BM, 16), lambda i: (i, 0))
    outs = ([jax.ShapeDtypeStruct((_N, _HPAD), jnp.float32)] * 2
            + [jax.ShapeDtypeStruct((_N, 16), jnp.float32)] * _NCHUNK)
    return pl.pallas_call(
        _prep_body,
        grid=(_NB,),
        in_specs=[bn_, bn_, bw, bb, bw, bb, bw, bd],
        out_specs=[bn_, bn_] + [bp] * _NCHUNK,
        out_shape=outs,
    )(x, emb, wn, bn, we, be, cw0, degslab)


def _post_body(acc, dref, h, wr, cb, t_out, sums):
    i = pl.program_id(0)
    dinv = _dinv_from_deg(dref)
    parts = [acc[k, 0] + acc[k, 1] for k in range(_NCHUNK)]
    agg = jnp.concatenate(
        parts + [jnp.zeros((_BM, _HPAD - 16 * _NCHUNK), jnp.float32)], axis=1)
    t = jnp.maximum(dinv * agg
                    + jnp.dot(h[...], wr[...],
                              preferred_element_type=jnp.float32) + cb[...],
                    0.0)
    t_out[...] = t

    @pl.when(i == 0)
    def _():
        sums[...] = jnp.zeros_like(sums)

    sums[0:1, :] = sums[0:1, :] + jnp.sum(t, axis=0, keepdims=True)
    sums[1:2, :] = sums[1:2, :] + jnp.sum(t * t, axis=0, keepdims=True)


def _tc_post(acc, degslab, h, wr, cb):
    bn_ = pl.BlockSpec((_BM, _HPAD), lambda i: (i, 0))
    ba = pl.BlockSpec((_NCHUNK, 2, _BM, 16), lambda i: (0, 0, i, 0))
    bd = pl.BlockSpec((2, _BM, 16), lambda i: (0, i, 0))
    bw = pl.BlockSpec((_HPAD, _HPAD), lambda i: (0, 0))
    bb = pl.BlockSpec((1, _HPAD), lambda i: (0, 0))
    bs = pl.BlockSpec((2, _HPAD), lambda i: (0, 0))
    return pl.pallas_call(
        _post_body,
        grid=(_NB,),
        in_specs=[ba, bd, bn_, bw, bb],
        out_specs=[bn_, bs],
        out_shape=[jax.ShapeDtypeStruct((_N, _HPAD), jnp.float32),
                   jax.ShapeDtypeStruct((2, _HPAD), jnp.float32)],
    )(acc, degslab, h, wr, cb)


def _bn_apply(t, sums, gamma, beta):
    mean = sums[0:1, :] * (1.0 / _N)
    ex2 = sums[1:2, :] * (1.0 / _N)
    var = ex2 - mean * mean
    inv = lax.rsqrt(var + 1e-5)
    return (t[...] - mean) * (inv * gamma[...]) + beta[...]


def _bnmm_body(t, sums, dref, gamma, beta, wnext, h_out, *pouts):
    h = _bn_apply(t, sums, gamma, beta)
    h_out[...] = h
    dinv = _dinv_from_deg(dref)
    outn = jnp.dot(h, wnext[...], preferred_element_type=jnp.float32)
    for k in range(_NCHUNK):
        pouts[k][...] = dinv * outn[:, 16 * k:16 * (k + 1)]


def _tc_bnmm(t, sums, degslab, gamma, beta, wnext):
    bn_ = pl.BlockSpec((_BM, _HPAD), lambda i: (i, 0))
    bs = pl.BlockSpec((2, _HPAD), lambda i: (0, 0))
    bd = pl.BlockSpec((2, _BM, 16), lambda i: (0, i, 0))
    bb = pl.BlockSpec((1, _HPAD), lambda i: (0, 0))
    bw = pl.BlockSpec((_HPAD, _HPAD), lambda i: (0, 0))
    bp = pl.BlockSpec((_BM, 16), lambda i: (i, 0))
    outs = ([jax.ShapeDtypeStruct((_N, _HPAD), jnp.float32)]
            + [jax.ShapeDtypeStruct((_N, 16), jnp.float32)] * _NCHUNK)
    return pl.pallas_call(
        _bnmm_body,
        grid=(_NB,),
        in_specs=[bn_, bs, bd, bb, bb, bw],
        out_specs=[bn_] + [bp] * _NCHUNK,
        out_shape=outs,
    )(t, sums, degslab, gamma, beta, wnext)


def _bnfin_body(t, sums, gamma, beta, e, wah, wae, wbh, wbe, uv):
    h = _bn_apply(t, sums, gamma, beta)
    u = (jnp.dot(h, wah[...], preferred_element_type=jnp.float32)
         + jnp.dot(e[...], wae[...], preferred_element_type=jnp.float32))
    v = (jnp.dot(h, wbh[...], preferred_element_type=jnp.float32)
         + jnp.dot(e[...], wbe[...], preferred_element_type=jnp.float32))
    uv[0, :, :] = u
    uv[1, :, :] = v


def _tc_bnfin(t, sums, gamma, beta, e, wah, wae, wbh, wbe):
    bn_ = pl.BlockSpec((_BM, _HPAD), lambda i: (i, 0))
    bs = pl.BlockSpec((2, _HPAD), lambda i: (0, 0))
    bb = pl.BlockSpec((1, _HPAD), lambda i: (0, 0))
    bw = pl.BlockSpec((_HPAD, _UPAD), lambda i: (0, 0))
    buv = pl.BlockSpec((2, _BM, _UPAD), lambda i: (0, i, 0))
    return pl.pallas_call(
        _bnfin_body,
        grid=(_NB,),
        in_specs=[bn_, bs, bb, bb, bn_, bw, bw, bw, bw],
        out_specs=buv,
        out_shape=jax.ShapeDtypeStruct((2, _N, _UPAD), jnp.float32),
    )(t, sums, gamma, beta, e, wah, wae, wbh, wbe)


def _head_body(g0, g1, b1p, w2p, b2p, out):
    tt = jnp.tanh(g0[...] + g1[...] + b1p[...])
    out[...] = jnp.sum(tt * w2p[...], axis=1, keepdims=True) + b2p[0, 0]


def _tc_head(gflat, b1p, w2p, b2p):
    bg0 = pl.BlockSpec((_BPE, _UPAD), lambda i: (i, 0))
    bg1 = pl.BlockSpec((_BPE, _UPAD), lambda i: (i + _NPB, 0))
    bb = pl.BlockSpec((1, _UPAD), lambda i: (0, 0))
    bsc = pl.BlockSpec((1, 1), lambda i: (0, 0))
    bo = pl.BlockSpec((_BPE, 1), lambda i: (i, 0))
    return pl.pallas_call(
        _head_body,
        grid=(_NPB,),
        in_specs=[bg0, bg1, bb, bb, bsc],
        out_specs=bo,
        out_shape=jax.ShapeDtypeStruct((_PE, 1), jnp.float32),
    )(gflat, gflat, b1p, w2p, b2p)


def kernel(x, emb, edge_index, pedge_index, W_node, b_node, W_emb, b_emb,
           conv_init_w, conv_root_w, conv_bias, bn_gamma, bn_beta,
           W1, b1, W2, b2):
    f32 = jnp.float32

    def padw(w, r, c):
        return jnp.pad(w, ((0, r - w.shape[0]), (0, c - w.shape[1])))

    def padv(v):
        return jnp.pad(v, (0, _HPAD - v.shape[0]))[None, :]

    wn = padw(W_node, _HPAD, _HPAD)
    we = padw(W_emb, _HPAD, _HPAD)
    bn = padv(b_node)
    be = padv(b_emb)
    cwi = [padw(conv_init_w[l], _HPAD, _HPAD) for l in range(3)]
    cwr = [padw(conv_root_w[l], _HPAD, _HPAD) for l in range(3)]
    cb = [padv(conv_bias[l]) for l in range(3)]
    gam = [padv(bn_gamma[l]) for l in range(3)]
    bet = [padv(bn_beta[l]) for l in range(3)]

    w1r = W1.reshape(2, 3, 2, _HID, _HID).sum(axis=1)
    wah = padw(w1r[0, 0], _HPAD, _UPAD)
    wae = padw(w1r[0, 1], _HPAD, _UPAD)
    wbh = padw(w1r[1, 0], _HPAD, _UPAD)
    wbe = padw(w1r[1, 1], _HPAD, _UPAD)
    b1p = jnp.pad(b1, (0, _UPAD - _HID))[None, :]
    w2p = jnp.pad(W2[:, 0], (0, _UPAD - _HID))[None, :]
    b2p = b2.reshape(1, 1)

    row = edge_index[0]
    col = edge_index[1]
    row2 = jnp.concatenate(
        [row, jnp.zeros((_EPAD - _E,), jnp.int32)]).reshape(_EPAD // 128, 128)
    col2 = jnp.concatenate(
        [col, jnp.full((_EPAD - _E,), _N, jnp.int32)]).reshape(_EPAD // 128, 128)
    idx2 = jnp.concatenate(
        [pedge_index[0], pedge_index[1] + _N,
         jnp.zeros((_GPAD - 2 * _PE,), jnp.int32)]).reshape(_GPAD // 128, 128)
    zslab = jnp.zeros((_TPW, 16), f32)
    ones16 = jnp.ones((128, 16), f32)

    degslab = _sc_degree(col2, ones16, zslab)
    h, e, *pls = _tc_prep(x, emb, wn, bn, we, be, cwi[0], degslab)
    uv = None
    for l in range(3):
        acc = _sc_segsum(pls, row2, col2, zslab)
        t, sums = _tc_post(acc, degslab, h, cwr[l], cb[l])
        if l < 2:
            h, *pls = _tc_bnmm(t, sums, degslab, gam[l], bet[l], cwi[l + 1])
        else:
            uv = _tc_bnfin(t, sums, gam[l], bet[l], e, wah, wae, wbh, wbe)

    g3 = _sc_gather(uv.reshape(2 * _N, _UPAD), idx2)
    return _tc_head(g3.reshape(_GPAD, _UPAD), b1p, w2p, b2p)
